# baseline (speedup 1.0000x reference)
import functools

import jax
import jax.numpy as jnp
from jax import lax
from jax.experimental import pallas as pl
from jax.experimental.pallas import tpu as pltpu
from jax.experimental.pallas import tpu_sc as plsc

B = 16384
EMB = 32
_CHUNK = 128


def _gather_body(ut, it, uidx, iidx, out_u, out_i, uidx_v, iidx_v, urows_v,
                 irows_v, sem_u, sem_i):
    info = plsc.get_sparse_core_info()
    nc = info.num_cores
    nw = nc * info.num_subcores
    bpw = B // nw
    nchunk = bpw // _CHUNK
    wid = lax.axis_index("s") * nc + lax.axis_index("c")
    base = wid * bpw
    for j in range(nchunk):
        pltpu.sync_copy(uidx.at[pl.ds(base + j * _CHUNK, _CHUNK)], uidx_v.at[j])
        pltpu.sync_copy(iidx.at[pl.ds(base + j * _CHUNK, _CHUNK)], iidx_v.at[j])
    copies = []
    for j in range(nchunk):
        copies.append(pltpu.async_copy(
            ut.at[uidx_v.at[j]], urows_v.at[pl.ds(j * _CHUNK, _CHUNK)], sem_u))
        copies.append(pltpu.async_copy(
            it.at[iidx_v.at[j]], irows_v.at[pl.ds(j * _CHUNK, _CHUNK)], sem_i))
    for c in copies:
        c.wait()
    pltpu.sync_copy(urows_v, out_u.at[pl.ds(base, bpw)])
    pltpu.sync_copy(irows_v, out_i.at[pl.ds(base, bpw)])


def _sc_gather(user_table, item_table, user_idx, item_idx):
    info = plsc.get_sparse_core_info()
    nw = info.num_cores * info.num_subcores
    bpw = B // nw
    nchunk = bpw // _CHUNK
    mesh = plsc.VectorSubcoreMesh(core_axis_name="c", subcore_axis_name="s")
    f = pl.kernel(
        _gather_body,
        mesh=mesh,
        compiler_params=pltpu.CompilerParams(use_tc_tiling_on_sc=False),
        out_type=[
            jax.ShapeDtypeStruct((B, EMB), jnp.float32),
            jax.ShapeDtypeStruct((B, EMB), jnp.float32),
        ],
        scratch_types=[
            pltpu.VMEM((nchunk, _CHUNK), jnp.int32),
            pltpu.VMEM((nchunk, _CHUNK), jnp.int32),
            pltpu.VMEM((bpw, EMB), jnp.float32),
            pltpu.VMEM((bpw, EMB), jnp.float32),
            pltpu.SemaphoreType.DMA,
            pltpu.SemaphoreType.DMA,
        ],
    )
    return f(user_table, item_table, user_idx, item_idx)


def _dense_body(xu_ref, xe_ref, xg_ref, uw1, ub1, uw2, ub2, uw3, ub3, gw, gb,
                iw1a, iw1b, ib1, iw2, ib2, iw3, ib3, out_ref):
    f32 = jnp.float32
    relu = lambda x: jnp.maximum(x, 0.0)
    dot = lambda a, b: jnp.dot(a, b, preferred_element_type=f32)
    h = relu(dot(xu_ref[...], uw1[...]) + ub1[...])
    h = relu(dot(h, uw2[...]) + ub2[...])
    u = dot(h, uw3[...]) + ub3[...]
    g = relu(dot(xg_ref[...], gw[...]) + gb[...])
    hi = relu(dot(xe_ref[...], iw1a[...]) + dot(g, iw1b[...]) + ib1[...])
    hi = relu(dot(hi, iw2[...]) + ib2[...])
    v = dot(hi, iw3[...]) + ib3[...]
    nu = jnp.maximum(jnp.sqrt(jnp.sum(u * u, axis=-1)), 1e-12)
    nv = jnp.maximum(jnp.sqrt(jnp.sum(v * v, axis=-1)), 1e-12)
    out_ref[...] = jnp.sum(u * v, axis=-1) / (nu * nv)


def _tc_dense(xu, xe, xg, u_w1, u_b1, u_w2, u_b2, u_w3, u_b3, g_w, g_b,
              i_w1a, i_w1b, i_b1, i_w2, i_b2, i_w3, i_b3):
    R = 2048
    grid = (B // R,)
    row_spec = lambda w: pl.BlockSpec((R, w), lambda i: (i, 0))
    full = lambda a: pl.BlockSpec(a.shape, lambda i: (0,) * a.ndim)
    return pl.pallas_call(
        _dense_body,
        grid=grid,
        in_specs=[
            row_spec(EMB), row_spec(EMB), row_spec(16),
            full(u_w1), full(u_b1), full(u_w2), full(u_b2), full(u_w3),
            full(u_b3), full(g_w), full(g_b), full(i_w1a), full(i_w1b),
            full(i_b1), full(i_w2), full(i_b2), full(i_w3), full(i_b3),
        ],
        out_specs=pl.BlockSpec((R,), lambda i: (i,)),
        out_shape=jax.ShapeDtypeStruct((B,), jnp.float32),
    )(xu, xe, xg, u_w1, u_b1, u_w2, u_b2, u_w3, u_b3, g_w, g_b,
      i_w1a, i_w1b, i_b1, i_w2, i_b2, i_w3, i_b3)


def kernel(user_idx, item_idx, genre_features, user_table, u_w1, u_b1, u_w2,
           u_b2, u_w3, u_b3, item_table, g_w, g_b, i_w1, i_b1, i_w2, i_b2,
           i_w3, i_b3):
    xu, xe = _sc_gather(user_table, item_table,
                        user_idx.astype(jnp.int32), item_idx.astype(jnp.int32))
    xg = jnp.pad(genre_features, ((0, 0), (0, 6)))
    gw = jnp.pad(g_w, ((0, 6), (0, 0)))
    r2 = lambda b: b.reshape(1, -1)
    return _tc_dense(xu, xe, xg, u_w1, r2(u_b1), u_w2, r2(u_b2), u_w3,
                     r2(u_b3), gw, r2(g_b), i_w1[:EMB], i_w1[EMB:], r2(i_b1),
                     i_w2, r2(i_b2), i_w3, r2(i_b3))
